# baseline (speedup 1.0000x reference)
import functools

import jax
import jax.numpy as jnp
from jax import lax
from jax.experimental import pallas as pl
from jax.experimental.pallas import tpu as pltpu
from jax.experimental.pallas import tpu_sc as plsc

B = 1
H = 32
L = 4096
D = 128

_NC = 2
_NS = 16
_NW = _NC * _NS
_ROWS = H * L
_RPW = _ROWS // _NW
_CH = 512
_NCHUNK = _RPW // _CH
_NBUF = 3

_mesh = plsc.VectorSubcoreMesh(core_axis_name="c", subcore_axis_name="s")


def _sc_copy_body(k_in, v_in, k_out, v_out, buf, rsem, wsem):
    wid = lax.axis_index("s") * _NC + lax.axis_index("c")
    base = wid * _RPW

    chunks = []
    for src, dst in ((k_in, k_out), (v_in, v_out)):
        for c in range(_NCHUNK):
            chunks.append((src, dst, base + c * _CH))
    n = len(chunks)

    reads = [None] * n
    writes = [None] * n

    def issue_read(i):
        src, _, off = chunks[i]
        b = i % _NBUF
        reads[i] = pltpu.async_copy(
            src.at[pl.ds(off, _CH)], buf.at[b], rsem.at[b])

    def issue_write(i):
        _, dst, off = chunks[i]
        b = i % _NBUF
        reads[i].wait()
        writes[i] = pltpu.async_copy(
            buf.at[b], dst.at[pl.ds(off, _CH)], wsem.at[b])

    for i in range(n):
        if i >= _NBUF:
            writes[i - _NBUF].wait()
        issue_read(i)
        j = i - (_NBUF - 1)
        if j >= 0:
            issue_write(j)
    for j in range(max(0, n - _NBUF + 1), n):
        issue_write(j)
    for j in range(max(0, n - _NBUF), n):
        writes[j].wait()


_sc_copy = functools.partial(
    pl.kernel,
    out_type=(
        jax.ShapeDtypeStruct((_ROWS, D), jnp.bfloat16),
        jax.ShapeDtypeStruct((_ROWS, D), jnp.bfloat16),
    ),
    mesh=_mesh,
    scratch_types=[
        pltpu.VMEM((_NBUF, _CH, D), jnp.bfloat16),
        pltpu.SemaphoreType.DMA((_NBUF,)),
        pltpu.SemaphoreType.DMA((_NBUF,)),
    ],
)(_sc_copy_body)


def kernel(input_pos, k_val, v_val):
    del input_pos
    k2 = k_val.reshape(_ROWS, D)
    v2 = v_val.reshape(_ROWS, D)
    k_out, v_out = _sc_copy(k2, v2)
    return (k_out.reshape(B, H, L, D), v_out.reshape(B, H, L, D))
